# Initial kernel scaffold; baseline (speedup 1.0000x reference)
import jax
import jax.numpy as jnp
from jax.experimental import pallas as pl


def kernel(user_id, user_hist, hist_mask, item_id, user_table, item_table):
    raise NotImplementedError("write your pallas kernel here")


# speedup vs baseline: 1.6928x; 1.6928x over previous
import jax
import jax.numpy as jnp
from jax import lax
from jax.experimental import pallas as pl
from jax.experimental.pallas import tpu as pltpu
from jax.experimental.pallas import tpu_sc as plsc

D = 32
B = 16384
L = 50
LP = 64

NC = 2
NS = 16
NW = NC * NS
BW = B // NW
C = 32
CHUNKS = BW // C
CL = C * L
G = 64
NG = CL // G


def _sc_body(user_id_hbm, hist_hbm, mask_hbm, item_id_hbm,
             user_table_hbm, item_table_hbm, out_hbm,
             idx_v, urows_v, irows_v, hidx_v, mask_v, rows_v, out_v, sem):
    wid = lax.axis_index("s") * NC + lax.axis_index("c")
    base = pl.multiple_of(wid * BW, BW)

    for table, src_ids, dst in ((user_table_hbm, user_id_hbm, urows_v),
                                (item_table_hbm, item_id_hbm, irows_v)):
        pltpu.sync_copy(src_ids.at[pl.ds(base, BW)], idx_v)
        cps = [
            pltpu.async_copy(table.at[idx_v.at[pl.ds(j * G, G)]],
                             dst.at[pl.ds(j * G, G)], sem)
            for j in range(BW // G)
        ]
        for cp in cps:
            cp.wait()

    def chunk_body(c, _):
        row0 = pl.multiple_of(base * L + c * CL, CL)
        m0 = pl.multiple_of(base * LP + c * C * LP, C * LP)
        pltpu.sync_copy(hist_hbm.at[pl.ds(row0, CL)], hidx_v)
        pltpu.sync_copy(mask_hbm.at[pl.ds(m0, C * LP)], mask_v)
        cps = [
            pltpu.async_copy(item_table_hbm.at[hidx_v.at[pl.ds(j * G, G)]],
                             rows_v.at[pl.ds(j * G, G)], sem)
            for j in range(NG)
        ]
        for cp in cps:
            cp.wait()

        def b_body(b, _):
            r0 = b * L
            mv = [mask_v[pl.ds(b * LP + 16 * k, 16)] for k in range(4)]
            acc = [jnp.zeros((16,), jnp.float32) for _ in range(8)]
            for l in range(L):
                m = mv[l // 16][l % 16]
                k = l % 4
                acc[2 * k] += m * rows_v[r0 + l, pl.ds(0, 16)]
                acc[2 * k + 1] += m * rows_v[r0 + l, pl.ds(16, 16)]
            scale = jnp.float32(1.0 / L)
            lo = (acc[0] + acc[2]) + (acc[4] + acc[6])
            hi = (acc[1] + acc[3]) + (acc[5] + acc[7])
            bb = c * C + b
            out_v[b, pl.ds(0, 16)] = urows_v[bb, pl.ds(0, 16)]
            out_v[b, pl.ds(16, 16)] = urows_v[bb, pl.ds(16, 16)]
            out_v[b, pl.ds(D, 16)] = lo * scale
            out_v[b, pl.ds(D + 16, 16)] = hi * scale
            out_v[b, pl.ds(2 * D, 16)] = irows_v[bb, pl.ds(0, 16)]
            out_v[b, pl.ds(2 * D + 16, 16)] = irows_v[bb, pl.ds(16, 16)]
            return 0

        lax.fori_loop(0, C, b_body, 0)
        pltpu.sync_copy(out_v, out_hbm.at[pl.ds(base + c * C, C), :])
        return 0

    lax.fori_loop(0, CHUNKS, chunk_body, 0)


@jax.jit
def _sc_call(user_id, hist_flat, mask_flat, item_id, user_table, item_table):
    mesh = plsc.VectorSubcoreMesh(core_axis_name="c", subcore_axis_name="s",
                                  num_cores=NC, num_subcores=NS)
    return pl.kernel(
        _sc_body,
        out_type=jax.ShapeDtypeStruct((B, 3 * D), jnp.float32),
        mesh=mesh,
        compiler_params=pltpu.CompilerParams(use_tc_tiling_on_sc=False),
        scratch_types=[
            pltpu.VMEM((BW,), jnp.int32),
            pltpu.VMEM((BW, D), jnp.float32),
            pltpu.VMEM((BW, D), jnp.float32),
            pltpu.VMEM((CL,), jnp.int32),
            pltpu.VMEM((C * LP,), jnp.float32),
            pltpu.VMEM((CL, D), jnp.float32),
            pltpu.VMEM((C, 3 * D), jnp.float32),
            pltpu.SemaphoreType.DMA,
        ],
    )(user_id, hist_flat, mask_flat, item_id, user_table, item_table)


def kernel(user_id, user_hist, hist_mask, item_id, user_table, item_table):
    user_id = user_id.astype(jnp.int32)
    item_id = item_id.astype(jnp.int32)
    hist_flat = user_hist.astype(jnp.int32).reshape(-1)
    mask_flat = jnp.pad(hist_mask, ((0, 0), (0, LP - L))).reshape(-1)
    return _sc_call(user_id, hist_flat, mask_flat, item_id,
                    user_table, item_table)


# speedup vs baseline: 1.7878x; 1.0561x over previous
import jax
import jax.numpy as jnp
from jax import lax
from jax.experimental import pallas as pl
from jax.experimental.pallas import tpu as pltpu
from jax.experimental.pallas import tpu_sc as plsc

VOCAB = 1000000
D = 32
B = 16384
L = 50
LP = 64

NC = 2
NS = 16
NW = NC * NS
BW = B // NW
C = 16
CHUNKS = BW // C
CL = C * L
G = 80
NG = CL // G
GU = 128


def _sc_body(user_id_hbm, hist_hbm, mask_hbm, item_id_hbm,
             user_table_hbm, item_table_hbm, out_hbm,
             uidx_v, iidx_v, urows_v, irows_v, hidx_v,
             mask_v0, mask_v1, rows_v0, rows_v1, out_v0, out_v1,
             sem_u, sem_g0, sem_g1, sem_m0, sem_m1, sem_o0, sem_o1):
    wid = lax.axis_index("s") * NC + lax.axis_index("c")
    base = pl.multiple_of(wid * BW, BW)
    rows_b = (rows_v0, rows_v1)
    mask_b = (mask_v0, mask_v1)
    out_b = (out_v0, out_v1)
    sem_g = (sem_g0, sem_g1)
    sem_m = (sem_m0, sem_m1)
    sem_o = (sem_o0, sem_o1)

    pltpu.sync_copy(hist_hbm.at[pl.ds(pl.multiple_of(base * L, BW * L),
                                      BW * L)], hidx_v)

    pltpu.sync_copy(user_id_hbm.at[pl.ds(base, BW)], uidx_v)
    pltpu.sync_copy(item_id_hbm.at[pl.ds(base, BW)], iidx_v)
    id_cps = []
    for j in range(BW // GU):
        id_cps.append(pltpu.async_copy(
            user_table_hbm.at[uidx_v.at[pl.ds(j * GU, GU)]],
            urows_v.at[pl.ds(j * GU, GU)], sem_u))
        id_cps.append(pltpu.async_copy(
            item_table_hbm.at[iidx_v.at[pl.ds(j * GU, GU)]],
            irows_v.at[pl.ds(j * GU, GU)], sem_u))

    def fire_gathers(t, buf):
        r0 = t * CL
        return [pltpu.async_copy(
            item_table_hbm.at[hidx_v.at[pl.ds(r0 + j * G, G)]],
            rows_b[buf].at[pl.ds(j * G, G)], sem_g[buf])
            for j in range(NG)]

    def mask_copy(t, buf):
        m0 = pl.multiple_of(base * LP, BW * LP) + t * (C * LP)
        return pltpu.make_async_copy(mask_hbm.at[pl.ds(m0, C * LP)],
                                     mask_b[buf], sem_m[buf])

    def out_copy(t, buf):
        return pltpu.make_async_copy(
            out_b[buf], out_hbm.at[pl.ds(base + t * C, C), :], sem_o[buf])

    fire_gathers(0, 0)
    mask_copy(0, 0).start()
    mask_copy(1, 1).start()
    for cp in id_cps:
        cp.wait()

    def compute_chunk(t, buf):
        rows_v, mask_v, out_v = rows_b[buf], mask_b[buf], out_b[buf]

        def b_body(b, _):
            r0 = b * L
            mv = [mask_v[pl.ds(b * LP + 16 * k, 16)] for k in range(4)]
            acc = [jnp.zeros((16,), jnp.float32) for _ in range(8)]
            for l in range(L):
                m = mv[l // 16][l % 16]
                k = l % 4
                acc[2 * k] += m * rows_v[r0 + l, pl.ds(0, 16)]
                acc[2 * k + 1] += m * rows_v[r0 + l, pl.ds(16, 16)]
            scale = jnp.float32(1.0 / L)
            lo = (acc[0] + acc[2]) + (acc[4] + acc[6])
            hi = (acc[1] + acc[3]) + (acc[5] + acc[7])
            bb = t * C + b
            out_v[b, pl.ds(0, 16)] = urows_v[bb, pl.ds(0, 16)]
            out_v[b, pl.ds(16, 16)] = urows_v[bb, pl.ds(16, 16)]
            out_v[b, pl.ds(D, 16)] = lo * scale
            out_v[b, pl.ds(D + 16, 16)] = hi * scale
            out_v[b, pl.ds(2 * D, 16)] = irows_v[bb, pl.ds(0, 16)]
            out_v[b, pl.ds(2 * D + 16, 16)] = irows_v[bb, pl.ds(16, 16)]
            return 0

        lax.fori_loop(0, C, b_body, 0)

    def wait_gathers(t, buf):
        r0 = t * CL
        for j in range(NG):
            pltpu.make_async_copy(
                item_table_hbm.at[hidx_v.at[pl.ds(r0 + j * G, G)]],
                rows_b[buf].at[pl.ds(j * G, G)], sem_g[buf]).wait()

    def loop_body(tt, _):
        for phase in range(2):
            t = tt * 2 + phase
            nxt = 1 - phase

            @pl.when(t + 1 < CHUNKS)
            def _():
                @pl.when(t + 1 >= 2)
                def _():
                    out_copy(t - 1, nxt).wait()
                fire_gathers(t + 1, nxt)

            wait_gathers(t, phase)
            mask_copy(t, phase).wait()
            compute_chunk(t, phase)
            out_copy(t, phase).start()

            @pl.when(t + 2 < CHUNKS)
            def _():
                mask_copy(t + 2, phase).start()
        return 0

    lax.fori_loop(0, CHUNKS // 2, loop_body, 0)
    out_copy(CHUNKS - 2, 0).wait()
    out_copy(CHUNKS - 1, 1).wait()


@jax.jit
def _sc_call(user_id, hist_flat, mask_flat, item_id, user_table, item_table):
    mesh = plsc.VectorSubcoreMesh(core_axis_name="c", subcore_axis_name="s",
                                  num_cores=NC, num_subcores=NS)
    return pl.kernel(
        _sc_body,
        out_type=jax.ShapeDtypeStruct((B, 3 * D), jnp.float32),
        mesh=mesh,
        compiler_params=pltpu.CompilerParams(use_tc_tiling_on_sc=False),
        scratch_types=[
            pltpu.VMEM((BW,), jnp.int32),
            pltpu.VMEM((BW,), jnp.int32),
            pltpu.VMEM((BW, D), jnp.float32),
            pltpu.VMEM((BW, D), jnp.float32),
            pltpu.VMEM((BW * L,), jnp.int32),
            pltpu.VMEM((C * LP,), jnp.float32),
            pltpu.VMEM((C * LP,), jnp.float32),
            pltpu.VMEM((CL, D), jnp.float32),
            pltpu.VMEM((CL, D), jnp.float32),
            pltpu.VMEM((C, 3 * D), jnp.float32),
            pltpu.VMEM((C, 3 * D), jnp.float32),
            pltpu.SemaphoreType.DMA,
            pltpu.SemaphoreType.DMA,
            pltpu.SemaphoreType.DMA,
            pltpu.SemaphoreType.DMA,
            pltpu.SemaphoreType.DMA,
            pltpu.SemaphoreType.DMA,
            pltpu.SemaphoreType.DMA,
        ],
    )(user_id, hist_flat, mask_flat, item_id, user_table, item_table)


def kernel(user_id, user_hist, hist_mask, item_id, user_table, item_table):
    user_id = user_id.astype(jnp.int32)
    item_id = item_id.astype(jnp.int32)
    hist_flat = user_hist.astype(jnp.int32).reshape(-1)
    mask_flat = jnp.pad(hist_mask, ((0, 0), (0, LP - L))).reshape(-1)
    return _sc_call(user_id, hist_flat, mask_flat, item_id,
                    user_table, item_table)
